# Initial kernel scaffold; baseline (speedup 1.0000x reference)
import jax
import jax.numpy as jnp
from jax.experimental import pallas as pl


def kernel(labels, weight):
    raise NotImplementedError("write your pallas kernel here")


# speedup vs baseline: 13.6649x; 13.6649x over previous
import functools

import jax
import jax.numpy as jnp
from jax import lax
from jax.experimental import pallas as pl
from jax.experimental.pallas import tpu as pltpu

_ROWS_PER_BLOCK = 8192


def _round_up(x, m):
    return ((x + m - 1) // m) * m


def _onehot_mxu_kernel(lbl_ref, w2_ref, o_ref, *, num_classes):
    tn = o_ref.shape[0]
    c2 = w2_ref.shape[0]
    lbl = lbl_ref[0]
    cls = lax.broadcasted_iota(jnp.int32, (c2, tn), 0) % num_classes
    oht = (cls == lbl).astype(jnp.bfloat16)
    o_ref[...] = lax.dot_general(
        oht, w2_ref[...],
        (((0,), (0,)), ((), ())),
        preferred_element_type=jnp.float32,
    )


def kernel(labels, weight):
    C, F = weight.shape
    orig_shape = labels.shape
    flat = labels.reshape(-1).astype(jnp.int32)
    N = flat.shape[0]

    tn = min(_ROWS_PER_BLOCK, _round_up(N, 8))
    n_pad = _round_up(N, tn)
    if n_pad != N:
        flat = jnp.pad(flat, (0, n_pad - N))
    g = n_pad // tn
    lbl3 = flat.reshape(g, 1, tn)

    w_hi = weight.astype(jnp.bfloat16)
    w_lo = (weight - w_hi.astype(jnp.float32)).astype(jnp.bfloat16)
    w2 = jnp.concatenate([w_hi, w_lo], axis=0)

    out = pl.pallas_call(
        functools.partial(_onehot_mxu_kernel, num_classes=C),
        out_shape=jax.ShapeDtypeStruct((n_pad, F), weight.dtype),
        grid=(g,),
        in_specs=[
            pl.BlockSpec((1, 1, tn), lambda i: (i, 0, 0)),
            pl.BlockSpec((2 * C, F), lambda i: (0, 0)),
        ],
        out_specs=pl.BlockSpec((tn, F), lambda i: (i, 0)),
        compiler_params=pltpu.CompilerParams(
            dimension_semantics=("parallel",),
        ),
    )(lbl3, w2)

    if n_pad != N:
        out = out[:N]
    return out.reshape(orig_shape + (F,))


# speedup vs baseline: 14.2500x; 1.0428x over previous
import functools

import jax
import jax.numpy as jnp
from jax import lax
from jax.experimental import pallas as pl
from jax.experimental.pallas import tpu as pltpu

_ROWS_PER_BLOCK = 16384


def _round_up(x, m):
    return ((x + m - 1) // m) * m


def _onehot_mxu_kernel(lbl_ref, w2_ref, o_ref, *, num_classes):
    tn = o_ref.shape[0]
    c2 = w2_ref.shape[0]
    lbl = lbl_ref[0]
    cls = lax.broadcasted_iota(jnp.int32, (c2, tn), 0) % num_classes
    oht = (cls == lbl).astype(jnp.bfloat16)
    o_ref[...] = lax.dot_general(
        oht, w2_ref[...],
        (((0,), (0,)), ((), ())),
        preferred_element_type=jnp.float32,
    )


def kernel(labels, weight):
    C, F = weight.shape
    orig_shape = labels.shape
    flat = labels.reshape(-1).astype(jnp.int32)
    N = flat.shape[0]

    tn = min(_ROWS_PER_BLOCK, _round_up(N, 8))
    n_pad = _round_up(N, tn)
    if n_pad != N:
        flat = jnp.pad(flat, (0, n_pad - N))
    g = n_pad // tn
    lbl3 = flat.reshape(g, 1, tn)

    w_hi32 = lax.reduce_precision(weight, exponent_bits=8, mantissa_bits=7)
    w_hi = w_hi32.astype(jnp.bfloat16)
    w_lo = (weight - w_hi32).astype(jnp.bfloat16)
    w2 = jnp.concatenate([w_hi, w_lo], axis=0)

    out = pl.pallas_call(
        functools.partial(_onehot_mxu_kernel, num_classes=C),
        out_shape=jax.ShapeDtypeStruct((n_pad, F), weight.dtype),
        grid=(g,),
        in_specs=[
            pl.BlockSpec((1, 1, tn), lambda i: (i, 0, 0)),
            pl.BlockSpec((2 * C, F), lambda i: (0, 0)),
        ],
        out_specs=pl.BlockSpec((tn, F), lambda i: (i, 0)),
        compiler_params=pltpu.CompilerParams(
            dimension_semantics=("parallel",),
        ),
    )(lbl3, w2)

    if n_pad != N:
        out = out[:N]
    return out.reshape(orig_shape + (F,))


# speedup vs baseline: 14.3189x; 1.0048x over previous
import functools

import jax
import jax.numpy as jnp
from jax import lax
from jax.experimental import pallas as pl
from jax.experimental.pallas import tpu as pltpu

_ROWS_PER_BLOCK = 32768


def _round_up(x, m):
    return ((x + m - 1) // m) * m


def _onehot_mxu_kernel(lbl_ref, w2_ref, o_ref, *, num_classes):
    tn = o_ref.shape[0]
    c2 = w2_ref.shape[0]
    lbl = lbl_ref[0]
    cls = lax.broadcasted_iota(jnp.int32, (c2, tn), 0) % num_classes
    oht = (cls == lbl).astype(jnp.bfloat16)
    o_ref[...] = lax.dot_general(
        oht, w2_ref[...],
        (((0,), (0,)), ((), ())),
        preferred_element_type=jnp.float32,
    )


def kernel(labels, weight):
    C, F = weight.shape
    orig_shape = labels.shape
    flat = labels.reshape(-1).astype(jnp.int32)
    N = flat.shape[0]

    tn = min(_ROWS_PER_BLOCK, _round_up(N, 8))
    n_pad = _round_up(N, tn)
    if n_pad != N:
        flat = jnp.pad(flat, (0, n_pad - N))
    g = n_pad // tn
    lbl3 = flat.reshape(g, 1, tn)

    w_hi32 = lax.reduce_precision(weight, exponent_bits=8, mantissa_bits=7)
    w_hi = w_hi32.astype(jnp.bfloat16)
    w_lo = (weight - w_hi32).astype(jnp.bfloat16)
    w2 = jnp.concatenate([w_hi, w_lo], axis=0)

    out = pl.pallas_call(
        functools.partial(_onehot_mxu_kernel, num_classes=C),
        out_shape=jax.ShapeDtypeStruct((n_pad, F), weight.dtype),
        grid=(g,),
        in_specs=[
            pl.BlockSpec((1, 1, tn), lambda i: (i, 0, 0)),
            pl.BlockSpec((2 * C, F), lambda i: (0, 0)),
        ],
        out_specs=pl.BlockSpec((tn, F), lambda i: (i, 0)),
        compiler_params=pltpu.CompilerParams(
            dimension_semantics=("parallel",),
        ),
    )(lbl3, w2)

    if n_pad != N:
        out = out[:N]
    return out.reshape(orig_shape + (F,))
